# baseline (speedup 1.0000x reference)
import functools

import jax
import jax.numpy as jnp
from jax import lax
from jax.experimental import pallas as pl
from jax.experimental.pallas import tpu as pltpu
from jax.experimental.pallas import tpu_sc as plsc

_BATCH = 16384
_DIM = 32
_NC = 2
_NS = 16
_NW = _NC * _NS
_BPW = _BATCH // _NW
_L = 16
_G = _BPW // _L
_ROWS = _BPW // 128


def _body(uid, iid, ut, it, out, uidx, iidx, urows, irows,
          outv, semu0, semi0, semu1, semi1):
    wid = lax.axis_index("s") * _NC + lax.axis_index("c")
    wrow = wid * _ROWS
    for r in range(_ROWS):
        pltpu.sync_copy(uid.at[wrow + r], uidx.at[pl.ds(r * 128, 128)])
        pltpu.sync_copy(iid.at[wrow + r], iidx.at[pl.ds(r * 128, 128)])

    lanes = lax.iota(jnp.int32, _L)
    sems = ((semu0, semi0), (semu1, semi1))

    def fetch_group(g, par):
        su, si = sems[par]
        uoff = lax.shift_left(uidx[pl.ds(g * _L, _L)], 5)
        ioff = lax.shift_left(iidx[pl.ds(g * _L, _L)], 5)
        for j in range(_L):
            slot = par * _L + j
            pltpu.async_copy(
                ut.at[pl.ds(pl.multiple_of(uoff[j], _DIM), _DIM)],
                urows.at[pl.ds(slot * _DIM, _DIM)], su)
            pltpu.async_copy(
                it.at[pl.ds(pl.multiple_of(ioff[j], _DIM), _DIM)],
                irows.at[pl.ds(slot * _DIM, _DIM)], si)

    def drain_group(par):
        su, si = sems[par]
        for j in range(_L):
            slot = par * _L + j
            pltpu.make_async_copy(
                ut.at[pl.ds(0, _DIM)],
                urows.at[pl.ds(slot * _DIM, _DIM)], su).wait()
            pltpu.make_async_copy(
                it.at[pl.ds(0, _DIM)],
                irows.at[pl.ds(slot * _DIM, _DIM)], si).wait()

    def compute_group(g, par):
        acc = jnp.zeros((_L,), jnp.float32)
        for j in range(_L):
            slot = par * _L + j
            u0 = urows[pl.ds(slot * _DIM, _L)]
            u1 = urows[pl.ds(slot * _DIM + _L, _L)]
            v0 = irows[pl.ds(slot * _DIM, _L)]
            v1 = irows[pl.ds(slot * _DIM + _L, _L)]
            s = u0 * v0 + u1 * v1
            acc = jnp.where(lanes == j, jnp.sum(s), acc)
        outv[pl.ds(g * _L, _L)] = acc

    def step(k, carry):
        g0 = 2 * k
        fetch_group(g0, 0)
        fetch_group(g0 + 1, 1)
        drain_group(0)
        compute_group(g0, 0)
        drain_group(1)
        compute_group(g0 + 1, 1)
        return carry

    lax.fori_loop(0, _G // 2, step, 0)

    for r in range(_ROWS):
        pltpu.sync_copy(outv.at[pl.ds(r * 128, 128)], out.at[wrow + r])


_course_rec = functools.partial(
    pl.kernel,
    out_type=jax.ShapeDtypeStruct((128, 128), jnp.float32),
    mesh=plsc.VectorSubcoreMesh(core_axis_name="c", subcore_axis_name="s"),
    compiler_params=pltpu.CompilerParams(
        needs_layout_passes=False, use_tc_tiling_on_sc=True
    ),
    scratch_types=[
        pltpu.VMEM((_BPW,), jnp.int32),
        pltpu.VMEM((_BPW,), jnp.int32),
        pltpu.VMEM((2 * _L * _DIM,), jnp.float32),
        pltpu.VMEM((2 * _L * _DIM,), jnp.float32),
        pltpu.VMEM((_BPW,), jnp.float32),
        pltpu.SemaphoreType.DMA,
        pltpu.SemaphoreType.DMA,
        pltpu.SemaphoreType.DMA,
        pltpu.SemaphoreType.DMA,
    ],
)(_body)


def kernel(user_ids, item_ids, user_table, item_table):
    uid = jnp.reshape(user_ids, (128, 128))
    iid = jnp.reshape(item_ids, (128, 128))
    ut = jnp.reshape(user_table, (-1,))
    it = jnp.reshape(item_table, (-1,))
    out = _course_rec(uid, iid, ut, it)
    return jnp.reshape(out, (_BATCH,))


# speedup vs baseline: 1.5700x; 1.5700x over previous
import functools

import jax
import jax.numpy as jnp
from jax import lax
from jax.experimental import pallas as pl
from jax.experimental.pallas import tpu as pltpu
from jax.experimental.pallas import tpu_sc as plsc

_BATCH = 16384
_DIM = 32
_NC = 2
_NS = 16
_NW = _NC * _NS
_BPW = _BATCH // _NW
_L = 16
_G = _BPW // _L
_ROWS = _BPW // 128


def _body(uid, iid, ut, it, out, uidx, iidx, urows, irows,
          outv, semu0, semi0, semu1, semi1):
    wid = lax.axis_index("s") * _NC + lax.axis_index("c")
    wrow = wid * _ROWS
    for r in range(_ROWS):
        pltpu.sync_copy(uid.at[wrow + r], uidx.at[pl.ds(r * 128, 128)])
        pltpu.sync_copy(iid.at[wrow + r], iidx.at[pl.ds(r * 128, 128)])

    lanes = lax.iota(jnp.int32, _L)
    sems = ((semu0, semi0), (semu1, semi1))

    def fetch_group(g, par):
        su, si = sems[par]
        uvec = uidx[pl.ds(g * _L, _L)]
        ivec = iidx[pl.ds(g * _L, _L)]
        for j in range(_L):
            slot = par * _L + j
            pltpu.async_copy(ut.at[pl.ds(uvec[j], 1)],
                             urows.at[pl.ds(slot, 1)], su)
            pltpu.async_copy(it.at[pl.ds(ivec[j], 1)],
                             irows.at[pl.ds(slot, 1)], si)

    def drain_group(par):
        su, si = sems[par]
        for j in range(_L):
            slot = par * _L + j
            pltpu.make_async_copy(ut.at[pl.ds(0, 1)],
                                  urows.at[pl.ds(slot, 1)], su).wait()
            pltpu.make_async_copy(it.at[pl.ds(0, 1)],
                                  irows.at[pl.ds(slot, 1)], si).wait()

    def compute_group(g, par):
        acc = jnp.zeros((_L,), jnp.float32)
        for j in range(_L):
            slot = par * _L + j
            u0 = urows[slot, pl.ds(0, _L)]
            u1 = urows[slot, pl.ds(_L, _L)]
            v0 = irows[slot, pl.ds(0, _L)]
            v1 = irows[slot, pl.ds(_L, _L)]
            s = u0 * v0 + u1 * v1
            acc = jnp.where(lanes == j, jnp.sum(s), acc)
        outv[pl.ds(g * _L, _L)] = acc

    def step(k, carry):
        g0 = 2 * k
        fetch_group(g0, 0)
        fetch_group(g0 + 1, 1)
        drain_group(0)
        compute_group(g0, 0)
        drain_group(1)
        compute_group(g0 + 1, 1)
        return carry

    lax.fori_loop(0, _G // 2, step, 0)

    for r in range(_ROWS):
        pltpu.sync_copy(outv.at[pl.ds(r * 128, 128)], out.at[wrow + r])


_course_rec = functools.partial(
    pl.kernel,
    out_type=jax.ShapeDtypeStruct((128, 128), jnp.float32),
    mesh=plsc.VectorSubcoreMesh(core_axis_name="c", subcore_axis_name="s"),
    compiler_params=pltpu.CompilerParams(needs_layout_passes=False),
    scratch_types=[
        pltpu.VMEM((_BPW,), jnp.int32),
        pltpu.VMEM((_BPW,), jnp.int32),
        pltpu.VMEM((2 * _L, _DIM), jnp.float32),
        pltpu.VMEM((2 * _L, _DIM), jnp.float32),
        pltpu.VMEM((_BPW,), jnp.float32),
        pltpu.SemaphoreType.DMA,
        pltpu.SemaphoreType.DMA,
        pltpu.SemaphoreType.DMA,
        pltpu.SemaphoreType.DMA,
    ],
)(_body)


def kernel(user_ids, item_ids, user_table, item_table):
    uid = jnp.reshape(user_ids, (128, 128))
    iid = jnp.reshape(item_ids, (128, 128))
    out = _course_rec(uid, iid, user_table, item_table)
    return jnp.reshape(out, (_BATCH,))
